# Initial kernel scaffold; baseline (speedup 1.0000x reference)
import jax
import jax.numpy as jnp
from jax.experimental import pallas as pl


def kernel(Ri, emb0_w0, emb0_b0, emb0_w1, emb0_b1, emb1_w0, emb1_b0, emb1_w1, emb1_b1, fit0_w0, fit0_b0, fit0_w1, fit0_b1, fit0_w2, fit0_b2, fit1_w0, fit1_b0, fit1_w1, fit1_b1, fit1_w2, fit1_b2):
    raise NotImplementedError("write your pallas kernel here")


# speedup vs baseline: 1.0020x; 1.0020x over previous
import functools

import jax
import jax.numpy as jnp
from jax import lax
from jax.experimental import pallas as pl
from jax.experimental.pallas import tpu as pltpu

NT = 2
M = 32
EH = 16
EE = 32
FH = 32
TA = 128
NNEI = NT * M
SEG = M * TA
SCALE = 1.0 / (M * NT)
DT = jnp.float32


def _tile_kernel(type_ids_ref, tile_ids_ref,
                 ri_ref,
                 ew0_ref, eb0_ref, ew1_ref, eb1_ref,
                 fw0_ref, fb0_ref, fw1_ref, fb1_ref, fw2_ref, fb2_ref,
                 ei_ref,
                 g_buf, scat_buf, dr_buf):
    del type_ids_ref, tile_ids_ref

    scat = tuple(jnp.zeros((EE, TA), jnp.float32) for _ in range(4))

    for t1 in range(NT):
        s = ri_ref[0, 0, 0:1, t1 * SEG:(t1 + 1) * SEG]
        h1 = jnp.tanh(ew0_ref[0, t1] * s + eb0_ref[0, t1])

        w1 = ew1_ref[0, t1]
        g = jnp.tanh(
            jnp.dot(w1, h1.astype(jnp.bfloat16),
                    preferred_element_type=jnp.float32) + eb1_ref[0, t1])
        g_buf[...] = g + jnp.concatenate([h1, h1], axis=0)

        def scat_body(n, carry, _t1=t1):
            s0, s1, s2, s3 = carry
            goff = pl.multiple_of(n * TA, TA)
            roff = pl.multiple_of(_t1 * SEG + n * TA, TA)
            gb = g_buf[:, pl.ds(goff, TA)]
            r = ri_ref[0, 0, :, pl.ds(roff, TA)]
            s0 = s0 + gb * r[0:1, :]
            s1 = s1 + gb * r[1:2, :]
            s2 = s2 + gb * r[2:3, :]
            s3 = s3 + gb * r[3:4, :]
            return (s0, s1, s2, s3)

        scat = lax.fori_loop(0, M, scat_body, scat, unroll=8)

    sb = []
    for c in range(4):
        sc = scat[c] * SCALE
        scat_buf[c * EE:(c + 1) * EE, :] = sc
        sb.append(sc[:EH, :])

    for e in range(EE):
        acc = scat_buf[e:e + 1, :] * sb[0]
        for c in range(1, 4):
            acc = acc + scat_buf[c * EE + e:c * EE + e + 1, :] * sb[c]
        dr_buf[e * EH:(e + 1) * EH, :] = acc.astype(jnp.bfloat16)

    dr = dr_buf[...]
    f1 = jnp.tanh(
        jnp.dot(fw0_ref[0], dr, preferred_element_type=jnp.float32)
        + fb0_ref[0])
    f2 = jnp.tanh(
        jnp.dot(fw1_ref[0], f1.astype(jnp.bfloat16),
                preferred_element_type=jnp.float32) + fb1_ref[0]) + f1
    ei = jnp.sum(f2 * fw2_ref[0], axis=0, keepdims=True) + fb2_ref[0]
    ei_ref[...] = ei.reshape(1, 1, 1, TA).astype(ei_ref.dtype)


def _pack_ri(Ri, natoms, n_tiles):
    B = Ri.shape[0]
    n_pad = n_tiles * TA
    packs = []
    lo = 0
    for t in range(NT):
        na = natoms[t]
        x = Ri[:, lo:lo + na].reshape(B * na, NNEI, 4)
        x = jnp.pad(x, ((0, n_pad - B * na), (0, 0), (0, 0)))
        x = x.transpose(2, 1, 0)
        x = x.reshape(4, NNEI, n_tiles, TA)
        x = x.transpose(2, 0, 1, 3).reshape(n_tiles, 4, NNEI * TA)
        packs.append(x)
        lo += na
    return jnp.stack(packs, axis=0)


def _pack_params(params):
    emb, fit = params['embedding'], params['fitting']
    ew0 = jnp.stack([jnp.transpose(emb[t]['w0'], (0, 2, 1)) for t in range(NT)])
    eb0 = jnp.stack([jnp.transpose(emb[t]['b0'], (0, 2, 1)) for t in range(NT)])
    ew1 = jnp.stack([jnp.transpose(emb[t]['w1'], (0, 2, 1))
                     for t in range(NT)]).astype(jnp.bfloat16)
    eb1 = jnp.stack([jnp.transpose(emb[t]['b1'], (0, 2, 1)) for t in range(NT)])
    fw0 = jnp.stack([fit[t]['w0'].T for t in range(NT)]).astype(jnp.bfloat16)
    fb0 = jnp.stack([fit[t]['b0'].T for t in range(NT)])
    fw1 = jnp.stack([fit[t]['w1'].T for t in range(NT)]).astype(jnp.bfloat16)
    fb1 = jnp.stack([fit[t]['b1'].T for t in range(NT)])
    fw2 = jnp.stack([fit[t]['w2'] for t in range(NT)])
    fb2 = jnp.stack([fit[t]['b2'] for t in range(NT)])
    return ew0, eb0, ew1, eb1, fw0, fb0, fw1, fb1, fw2, fb2


def _run(type_ids, tile_ids, ri_packed, weights, n_tiles, n_steps):
    def wspec(shape):
        nd = len(shape)
        return pl.BlockSpec((1,) + tuple(shape[1:]),
                            lambda i, tt, ti, _nd=nd: (tt[i],) + (0,) * (_nd - 1))

    in_specs = [pl.BlockSpec((1, 1, 4, NNEI * TA),
                             lambda i, tt, ti: (tt[i], ti[i], 0, 0))]
    in_specs += [wspec(w.shape) for w in weights]

    return pl.pallas_call(
        _tile_kernel,
        out_shape=jax.ShapeDtypeStruct((NT, n_tiles, 1, TA), DT),
        grid_spec=pltpu.PrefetchScalarGridSpec(
            num_scalar_prefetch=2,
            grid=(n_steps,),
            in_specs=in_specs,
            out_specs=pl.BlockSpec((1, 1, 1, TA),
                                   lambda i, tt, ti: (tt[i], ti[i], 0, 0)),
            scratch_shapes=[
                pltpu.VMEM((EE, SEG), jnp.float32),
                pltpu.VMEM((4 * EE, TA), jnp.float32),
                pltpu.VMEM((EH * EE, TA), jnp.bfloat16),
            ],
        ),
        compiler_params=pltpu.CompilerParams(
            dimension_semantics=("parallel",),
            vmem_limit_bytes=32 * 1024 * 1024,
        ),
    )(type_ids, tile_ids, ri_packed, *weights)


@functools.partial(jax.jit, static_argnums=(1,))
def _forward(Ri, natoms, params):
    B = Ri.shape[0]
    counts = [B * n for n in natoms]
    tiles = [pl.cdiv(c, TA) for c in counts]
    n_tiles = max(tiles)

    ri_packed = _pack_ri(Ri, natoms, n_tiles)
    weights = _pack_params(params)

    type_ids, tile_ids = [], []
    for t in range(NT):
        for i in range(tiles[t]):
            type_ids.append(t)
            tile_ids.append(i)
    n_steps = len(type_ids)
    type_ids = jnp.asarray(type_ids, jnp.int32)
    tile_ids = jnp.asarray(tile_ids, jnp.int32)

    ei_raw = _run(type_ids, tile_ids, ri_packed, weights, n_tiles, n_steps)

    ei_parts = []
    for t in range(NT):
        flat = ei_raw[t].reshape(n_tiles * TA)[:counts[t]]
        ei_parts.append(flat.reshape(B, natoms[t]))
    Ei = jnp.concatenate(ei_parts, axis=1)
    Etot = jnp.sum(Ei, axis=1, keepdims=True)
    F = jnp.zeros((B, sum(natoms), 3), DT)
    return Etot, Ei, F


def kernel(Ri,
           emb0_w0, emb0_b0, emb0_w1, emb0_b1,
           emb1_w0, emb1_b0, emb1_w1, emb1_b1,
           fit0_w0, fit0_b0, fit0_w1, fit0_b1, fit0_w2, fit0_b2,
           fit1_w0, fit1_b0, fit1_w1, fit1_b1, fit1_w2, fit1_b2):
    params = {
        'embedding': [
            {'w0': emb0_w0, 'b0': emb0_b0, 'w1': emb0_w1, 'b1': emb0_b1},
            {'w0': emb1_w0, 'b0': emb1_b0, 'w1': emb1_w1, 'b1': emb1_b1},
        ],
        'fitting': [
            {'w0': fit0_w0, 'b0': fit0_b0, 'w1': fit0_w1, 'b1': fit0_b1,
             'w2': fit0_w2, 'b2': fit0_b2},
            {'w0': fit1_w0, 'b0': fit1_b0, 'w1': fit1_w1, 'b1': fit1_b1,
             'w2': fit1_w2, 'b2': fit1_b2},
        ],
    }
    return _forward(Ri, (2048, 2048), params)


# speedup vs baseline: 1.2327x; 1.2303x over previous
import functools

import jax
import jax.numpy as jnp
from jax import lax
from jax.experimental import pallas as pl
from jax.experimental.pallas import tpu as pltpu

NT = 2
M = 32
EH = 16
EE = 32
FH = 32
TA = 128
NNEI = NT * M
NC = NNEI * 4
SEG = M * TA
SCALE = 1.0 / (M * NT)
DT = jnp.float32


def _tile_kernel(type_ids_ref,
                 rif_ref,
                 ew0_ref, eb0_ref, ew1_ref, eb1_ref,
                 fw0_ref, fb0_ref, fw1_ref, fb1_ref, fw2_ref, fb2_ref,
                 ei_ref,
                 rt_buf, s_buf, g_buf, scat_buf, dr_buf):
    del type_ids_ref

    rt_buf[...] = rif_ref[0].T

    for t1 in range(NT):
        for n in range(M):
            row = 4 * (M * t1 + n)
            s_buf[t1:t1 + 1, n * TA:(n + 1) * TA] = rt_buf[row:row + 1, :]

    scat = tuple(jnp.zeros((EE, TA), jnp.float32) for _ in range(4))

    for t1 in range(NT):
        s = s_buf[t1:t1 + 1, :]
        h1 = jnp.tanh(ew0_ref[0, t1] * s + eb0_ref[0, t1])

        g = jnp.tanh(
            jnp.dot(ew1_ref[0, t1], h1.astype(jnp.bfloat16),
                    preferred_element_type=jnp.float32) + eb1_ref[0, t1])
        g_buf[...] = g + jnp.concatenate([h1, h1], axis=0)

        def scat_body(n, carry, _t1=t1):
            s0, s1, s2, s3 = carry
            goff = pl.multiple_of(n * TA, TA)
            gb = g_buf[:, pl.ds(goff, TA)]
            roff = pl.multiple_of(4 * (M * _t1) + 4 * n, 4)
            r = rt_buf[pl.ds(roff, 4), :]
            s0 = s0 + gb * r[0:1, :]
            s1 = s1 + gb * r[1:2, :]
            s2 = s2 + gb * r[2:3, :]
            s3 = s3 + gb * r[3:4, :]
            return (s0, s1, s2, s3)

        scat = lax.fori_loop(0, M, scat_body, scat, unroll=8)

    sb = []
    for c in range(4):
        sc = scat[c] * SCALE
        scat_buf[c * EE:(c + 1) * EE, :] = sc
        sb.append(sc[:EH, :])

    for e in range(EE):
        acc = scat_buf[e:e + 1, :] * sb[0]
        for c in range(1, 4):
            acc = acc + scat_buf[c * EE + e:c * EE + e + 1, :] * sb[c]
        dr_buf[e * EH:(e + 1) * EH, :] = acc.astype(jnp.bfloat16)

    dr = dr_buf[...]
    f1 = jnp.tanh(
        jnp.dot(fw0_ref[0], dr, preferred_element_type=jnp.float32)
        + fb0_ref[0])
    f2 = jnp.tanh(
        jnp.dot(fw1_ref[0], f1.astype(jnp.bfloat16),
                preferred_element_type=jnp.float32) + fb1_ref[0]) + f1
    ei = jnp.sum(f2 * fw2_ref[0], axis=0, keepdims=True) + fb2_ref[0]
    ei_ref[...] = ei.reshape(1, 1, TA).astype(ei_ref.dtype)


def _pack_params(params):
    emb, fit = params['embedding'], params['fitting']
    ew0 = jnp.stack([jnp.transpose(emb[t]['w0'], (0, 2, 1)) for t in range(NT)])
    eb0 = jnp.stack([jnp.transpose(emb[t]['b0'], (0, 2, 1)) for t in range(NT)])
    ew1 = jnp.stack([jnp.transpose(emb[t]['w1'], (0, 2, 1))
                     for t in range(NT)]).astype(jnp.bfloat16)
    eb1 = jnp.stack([jnp.transpose(emb[t]['b1'], (0, 2, 1)) for t in range(NT)])
    fw0 = jnp.stack([fit[t]['w0'].T for t in range(NT)]).astype(jnp.bfloat16)
    fb0 = jnp.stack([fit[t]['b0'].T for t in range(NT)])
    fw1 = jnp.stack([fit[t]['w1'].T for t in range(NT)]).astype(jnp.bfloat16)
    fb1 = jnp.stack([fit[t]['b1'].T for t in range(NT)])
    fw2 = jnp.stack([fit[t]['w2'] for t in range(NT)])
    fb2 = jnp.stack([fit[t]['b2'] for t in range(NT)])
    return ew0, eb0, ew1, eb1, fw0, fb0, fw1, fb1, fw2, fb2


def _run(type_ids, rif, weights, n_steps):
    def wspec(shape):
        nd = len(shape)
        return pl.BlockSpec((1,) + tuple(shape[1:]),
                            lambda i, tt, _nd=nd: (tt[i],) + (0,) * (_nd - 1))

    in_specs = [pl.BlockSpec((1, TA, NC), lambda i, tt: (i, 0, 0))]
    in_specs += [wspec(w.shape) for w in weights]

    return pl.pallas_call(
        _tile_kernel,
        out_shape=jax.ShapeDtypeStruct((n_steps, 1, TA), DT),
        grid_spec=pltpu.PrefetchScalarGridSpec(
            num_scalar_prefetch=1,
            grid=(n_steps,),
            in_specs=in_specs,
            out_specs=pl.BlockSpec((1, 1, TA), lambda i, tt: (i, 0, 0)),
            scratch_shapes=[
                pltpu.VMEM((NC, TA), jnp.float32),
                pltpu.VMEM((NT, SEG), jnp.float32),
                pltpu.VMEM((EE, SEG), jnp.float32),
                pltpu.VMEM((4 * EE, TA), jnp.float32),
                pltpu.VMEM((EH * EE, TA), jnp.bfloat16),
            ],
        ),
        compiler_params=pltpu.CompilerParams(
            dimension_semantics=("parallel",),
            vmem_limit_bytes=32 * 1024 * 1024,
        ),
    )(type_ids, rif, *weights)


@functools.partial(jax.jit, static_argnums=(1,))
def _forward(Ri, natoms, params):
    B = Ri.shape[0]
    natoms_sum = sum(natoms)
    rows = B * natoms_sum
    n_steps = rows // TA
    rif = Ri.reshape(rows // TA, TA, NC)

    weights = _pack_params(params)

    type_ids = []
    for _ in range(B):
        for t in range(NT):
            type_ids += [t] * (natoms[t] // TA)
    type_ids = jnp.asarray(type_ids, jnp.int32)

    ei_raw = _run(type_ids, rif, weights, n_steps)

    Ei = ei_raw.reshape(B, natoms_sum)
    Etot = jnp.sum(Ei, axis=1, keepdims=True)
    F = jnp.zeros((B, natoms_sum, 3), DT)
    return Etot, Ei, F


def kernel(Ri,
           emb0_w0, emb0_b0, emb0_w1, emb0_b1,
           emb1_w0, emb1_b0, emb1_w1, emb1_b1,
           fit0_w0, fit0_b0, fit0_w1, fit0_b1, fit0_w2, fit0_b2,
           fit1_w0, fit1_b0, fit1_w1, fit1_b1, fit1_w2, fit1_b2):
    params = {
        'embedding': [
            {'w0': emb0_w0, 'b0': emb0_b0, 'w1': emb0_w1, 'b1': emb0_b1},
            {'w0': emb1_w0, 'b0': emb1_b0, 'w1': emb1_w1, 'b1': emb1_b1},
        ],
        'fitting': [
            {'w0': fit0_w0, 'b0': fit0_b0, 'w1': fit0_w1, 'b1': fit0_b1,
             'w2': fit0_w2, 'b2': fit0_b2},
            {'w0': fit1_w0, 'b0': fit1_b0, 'w1': fit1_w1, 'b1': fit1_b1,
             'w2': fit1_w2, 'b2': fit1_b2},
        ],
    }
    return _forward(Ri, (2048, 2048), params)


# speedup vs baseline: 1.4836x; 1.2035x over previous
import functools

import jax
import jax.numpy as jnp
from jax import lax
from jax.experimental import pallas as pl
from jax.experimental.pallas import tpu as pltpu

NT = 2
M = 32
EH = 16
EE = 32
FH = 32
TA = 128
NNEI = NT * M
NC = NNEI * 4
SEG = M * TA
SCALE = 1.0 / (M * NT)
DT = jnp.float32


def _tile_kernel(type_ids_ref,
                 rif_ref,
                 ew0_ref, eb0_ref, ew1_ref, eb1_ref,
                 fw0_ref, fb0_ref, fw1_ref, fb1_ref, fw2_ref, fb2_ref,
                 ei_ref,
                 rt_buf, s_buf, g0_buf, g1_buf, scat_buf, dr_buf):
    del type_ids_ref

    rt_buf[...] = rif_ref[0].T

    for t1 in range(NT):
        for n in range(M):
            row = 4 * (M * t1 + n)
            s_buf[t1:t1 + 1, n * TA:(n + 1) * TA] = rt_buf[row:row + 1, :]

    gbufs = (g0_buf, g1_buf)
    for t1 in range(NT):
        s = s_buf[t1:t1 + 1, :]
        h1 = jnp.tanh(ew0_ref[0, t1] * s + eb0_ref[0, t1])
        g = jnp.tanh(
            jnp.dot(ew1_ref[0, t1], h1.astype(jnp.bfloat16),
                    preferred_element_type=jnp.float32) + eb1_ref[0, t1])
        gbufs[t1][...] = g + jnp.concatenate([h1, h1], axis=0)

    scat = tuple(jnp.zeros((EE, TA), jnp.float32) for _ in range(4))
    for t1 in range(NT):
        for n in range(M):
            gb = gbufs[t1][:, n * TA:(n + 1) * TA]
            r = rt_buf[4 * (M * t1 + n):4 * (M * t1 + n) + 4, :]
            scat = tuple(scat[c] + gb * r[c:c + 1, :] for c in range(4))

    sb = []
    for c in range(4):
        sc = scat[c] * SCALE
        scat_buf[c * EE:(c + 1) * EE, :] = sc
        sb.append(sc[:EH, :])

    for e in range(EE):
        acc = scat_buf[e:e + 1, :] * sb[0]
        for c in range(1, 4):
            acc = acc + scat_buf[c * EE + e:c * EE + e + 1, :] * sb[c]
        dr_buf[e * EH:(e + 1) * EH, :] = acc.astype(jnp.bfloat16)

    dr = dr_buf[...]
    f1 = jnp.tanh(
        jnp.dot(fw0_ref[0], dr, preferred_element_type=jnp.float32)
        + fb0_ref[0])
    f2 = jnp.tanh(
        jnp.dot(fw1_ref[0], f1.astype(jnp.bfloat16),
                preferred_element_type=jnp.float32) + fb1_ref[0]) + f1
    ei = jnp.sum(f2 * fw2_ref[0], axis=0, keepdims=True) + fb2_ref[0]
    ei_ref[...] = ei.reshape(1, 1, TA).astype(ei_ref.dtype)


def _pack_params(params):
    emb, fit = params['embedding'], params['fitting']
    ew0 = jnp.stack([jnp.transpose(emb[t]['w0'], (0, 2, 1)) for t in range(NT)])
    eb0 = jnp.stack([jnp.transpose(emb[t]['b0'], (0, 2, 1)) for t in range(NT)])
    ew1 = jnp.stack([jnp.transpose(emb[t]['w1'], (0, 2, 1))
                     for t in range(NT)]).astype(jnp.bfloat16)
    eb1 = jnp.stack([jnp.transpose(emb[t]['b1'], (0, 2, 1)) for t in range(NT)])
    fw0 = jnp.stack([fit[t]['w0'].T for t in range(NT)]).astype(jnp.bfloat16)
    fb0 = jnp.stack([fit[t]['b0'].T for t in range(NT)])
    fw1 = jnp.stack([fit[t]['w1'].T for t in range(NT)]).astype(jnp.bfloat16)
    fb1 = jnp.stack([fit[t]['b1'].T for t in range(NT)])
    fw2 = jnp.stack([fit[t]['w2'] for t in range(NT)])
    fb2 = jnp.stack([fit[t]['b2'] for t in range(NT)])
    return ew0, eb0, ew1, eb1, fw0, fb0, fw1, fb1, fw2, fb2


def _run(type_ids, rif, weights, n_steps):
    def wspec(shape):
        nd = len(shape)
        return pl.BlockSpec((1,) + tuple(shape[1:]),
                            lambda i, tt, _nd=nd: (tt[i],) + (0,) * (_nd - 1))

    in_specs = [pl.BlockSpec((1, TA, NC), lambda i, tt: (i, 0, 0))]
    in_specs += [wspec(w.shape) for w in weights]

    return pl.pallas_call(
        _tile_kernel,
        out_shape=jax.ShapeDtypeStruct((n_steps, 1, TA), DT),
        grid_spec=pltpu.PrefetchScalarGridSpec(
            num_scalar_prefetch=1,
            grid=(n_steps,),
            in_specs=in_specs,
            out_specs=pl.BlockSpec((1, 1, TA), lambda i, tt: (i, 0, 0)),
            scratch_shapes=[
                pltpu.VMEM((NC, TA), jnp.float32),
                pltpu.VMEM((NT, SEG), jnp.float32),
                pltpu.VMEM((EE, SEG), jnp.float32),
                pltpu.VMEM((EE, SEG), jnp.float32),
                pltpu.VMEM((4 * EE, TA), jnp.float32),
                pltpu.VMEM((EH * EE, TA), jnp.bfloat16),
            ],
        ),
        compiler_params=pltpu.CompilerParams(
            dimension_semantics=("parallel",),
            vmem_limit_bytes=32 * 1024 * 1024,
        ),
    )(type_ids, rif, *weights)


@functools.partial(jax.jit, static_argnums=(1,))
def _forward(Ri, natoms, params):
    B = Ri.shape[0]
    natoms_sum = sum(natoms)
    rows = B * natoms_sum
    n_steps = rows // TA
    rif = Ri.reshape(rows // TA, TA, NC)

    weights = _pack_params(params)

    type_ids = []
    for _ in range(B):
        for t in range(NT):
            type_ids += [t] * (natoms[t] // TA)
    type_ids = jnp.asarray(type_ids, jnp.int32)

    ei_raw = _run(type_ids, rif, weights, n_steps)

    Ei = ei_raw.reshape(B, natoms_sum)
    Etot = jnp.sum(Ei, axis=1, keepdims=True)
    F = jnp.zeros((B, natoms_sum, 3), DT)
    return Etot, Ei, F


def kernel(Ri,
           emb0_w0, emb0_b0, emb0_w1, emb0_b1,
           emb1_w0, emb1_b0, emb1_w1, emb1_b1,
           fit0_w0, fit0_b0, fit0_w1, fit0_b1, fit0_w2, fit0_b2,
           fit1_w0, fit1_b0, fit1_w1, fit1_b1, fit1_w2, fit1_b2):
    params = {
        'embedding': [
            {'w0': emb0_w0, 'b0': emb0_b0, 'w1': emb0_w1, 'b1': emb0_b1},
            {'w0': emb1_w0, 'b0': emb1_b0, 'w1': emb1_w1, 'b1': emb1_b1},
        ],
        'fitting': [
            {'w0': fit0_w0, 'b0': fit0_b0, 'w1': fit0_w1, 'b1': fit0_b1,
             'w2': fit0_w2, 'b2': fit0_b2},
            {'w0': fit1_w0, 'b0': fit1_b0, 'w1': fit1_w1, 'b1': fit1_b1,
             'w2': fit1_w2, 'b2': fit1_b2},
        ],
    }
    return _forward(Ri, (2048, 2048), params)


# speedup vs baseline: 1.6879x; 1.1377x over previous
import functools

import jax
import jax.numpy as jnp
from jax.experimental import pallas as pl
from jax.experimental.pallas import tpu as pltpu

NT = 2
M = 32
EH = 16
EE = 32
FH = 32
TA = 128
TPS = 2
NNEI = NT * M
NC = NNEI * 4
SEG = M * TA
SCALE = 1.0 / (M * NT)
DT = jnp.float32


def _tile_kernel(type_ids_ref,
                 rif_ref,
                 ew0_ref, eb0_ref, ew1_ref, eb1_ref,
                 fw0_ref, fb0_ref, fw1_ref, fb1_ref, fw2_ref, fb2_ref,
                 ei_ref,
                 rt_buf, s_buf, g_buf, scat_buf, dr_buf):
    del type_ids_ref

    for tile in range(TPS):
        rt_buf[tile] = rif_ref[tile].T

        for t1 in range(NT):
            for n in range(M):
                row = 4 * (M * t1 + n)
                s_buf[tile, t1:t1 + 1, n * TA:(n + 1) * TA] = \
                    rt_buf[tile, row:row + 1, :]

        for t1 in range(NT):
            s = s_buf[tile, t1:t1 + 1, :]
            h1 = jnp.tanh(ew0_ref[0, t1] * s + eb0_ref[0, t1])
            g = jnp.tanh(
                jnp.dot(ew1_ref[0, t1], h1.astype(jnp.bfloat16),
                        preferred_element_type=jnp.float32) + eb1_ref[0, t1])
            g_buf[tile, t1] = g + jnp.concatenate([h1, h1], axis=0)

        scat = tuple(jnp.zeros((EE, TA), jnp.float32) for _ in range(4))
        for t1 in range(NT):
            for n in range(M):
                gb = g_buf[tile, t1, :, n * TA:(n + 1) * TA]
                row = 4 * (M * t1 + n)
                r = rt_buf[tile, row:row + 4, :]
                scat = tuple(scat[c] + gb * r[c:c + 1, :] for c in range(4))

        sb = []
        for c in range(4):
            scat_buf[tile, c * EE:(c + 1) * EE, :] = scat[c]
            sb.append(scat[c][:EH, :])

        for e in range(EE):
            acc = scat_buf[tile, e:e + 1, :] * sb[0]
            for c in range(1, 4):
                acc = acc + scat_buf[tile, c * EE + e:c * EE + e + 1, :] * sb[c]
            dr_buf[tile, e * EH:(e + 1) * EH, :] = acc.astype(jnp.bfloat16)

        dr = dr_buf[tile]
        f1 = jnp.tanh(
            jnp.dot(fw0_ref[0], dr, preferred_element_type=jnp.float32)
            + fb0_ref[0])
        f2 = jnp.tanh(
            jnp.dot(fw1_ref[0], f1.astype(jnp.bfloat16),
                    preferred_element_type=jnp.float32) + fb1_ref[0]) + f1
        ei = jnp.sum(f2 * fw2_ref[0], axis=0, keepdims=True) + fb2_ref[0]
        ei_ref[tile] = ei.reshape(1, TA).astype(ei_ref.dtype)


def _pack_params(params):
    emb, fit = params['embedding'], params['fitting']
    ew0 = jnp.stack([jnp.transpose(emb[t]['w0'], (0, 2, 1)) for t in range(NT)])
    eb0 = jnp.stack([jnp.transpose(emb[t]['b0'], (0, 2, 1)) for t in range(NT)])
    ew1 = jnp.stack([jnp.transpose(emb[t]['w1'], (0, 2, 1))
                     for t in range(NT)]).astype(jnp.bfloat16)
    eb1 = jnp.stack([jnp.transpose(emb[t]['b1'], (0, 2, 1)) for t in range(NT)])
    fw0 = jnp.stack([(SCALE * SCALE) * fit[t]['w0'].T
                     for t in range(NT)]).astype(jnp.bfloat16)
    fb0 = jnp.stack([fit[t]['b0'].T for t in range(NT)])
    fw1 = jnp.stack([fit[t]['w1'].T for t in range(NT)]).astype(jnp.bfloat16)
    fb1 = jnp.stack([fit[t]['b1'].T for t in range(NT)])
    fw2 = jnp.stack([fit[t]['w2'] for t in range(NT)])
    fb2 = jnp.stack([fit[t]['b2'] for t in range(NT)])
    return ew0, eb0, ew1, eb1, fw0, fb0, fw1, fb1, fw2, fb2


def _run(type_ids, rif, weights, n_steps):
    def wspec(shape):
        nd = len(shape)
        return pl.BlockSpec((1,) + tuple(shape[1:]),
                            lambda i, tt, _nd=nd: (tt[i],) + (0,) * (_nd - 1))

    in_specs = [pl.BlockSpec((TPS, TA, NC), lambda i, tt: (i, 0, 0))]
    in_specs += [wspec(w.shape) for w in weights]

    return pl.pallas_call(
        _tile_kernel,
        out_shape=jax.ShapeDtypeStruct((n_steps * TPS, 1, TA), DT),
        grid_spec=pltpu.PrefetchScalarGridSpec(
            num_scalar_prefetch=1,
            grid=(n_steps,),
            in_specs=in_specs,
            out_specs=pl.BlockSpec((TPS, 1, TA), lambda i, tt: (i, 0, 0)),
            scratch_shapes=[
                pltpu.VMEM((TPS, NC, TA), jnp.float32),
                pltpu.VMEM((TPS, NT, SEG), jnp.float32),
                pltpu.VMEM((TPS, NT, EE, SEG), jnp.float32),
                pltpu.VMEM((TPS, 4 * EE, TA), jnp.float32),
                pltpu.VMEM((TPS, EH * EE, TA), jnp.bfloat16),
            ],
        ),
        compiler_params=pltpu.CompilerParams(
            dimension_semantics=("parallel",),
            vmem_limit_bytes=32 * 1024 * 1024,
        ),
    )(type_ids, rif, *weights)


@functools.partial(jax.jit, static_argnums=(1,))
def _forward(Ri, natoms, params):
    B = Ri.shape[0]
    natoms_sum = sum(natoms)
    rows = B * natoms_sum
    n_steps = rows // (TA * TPS)
    rif = Ri.reshape(rows // TA, TA, NC)

    weights = _pack_params(params)

    type_ids = []
    for _ in range(B):
        for t in range(NT):
            type_ids += [t] * (natoms[t] // (TA * TPS))
    type_ids = jnp.asarray(type_ids, jnp.int32)

    ei_raw = _run(type_ids, rif, weights, n_steps)

    Ei = ei_raw.reshape(B, natoms_sum)
    Etot = jnp.sum(Ei, axis=1, keepdims=True)
    F = jnp.zeros((B, natoms_sum, 3), DT)
    return Etot, Ei, F


def kernel(Ri,
           emb0_w0, emb0_b0, emb0_w1, emb0_b1,
           emb1_w0, emb1_b0, emb1_w1, emb1_b1,
           fit0_w0, fit0_b0, fit0_w1, fit0_b1, fit0_w2, fit0_b2,
           fit1_w0, fit1_b0, fit1_w1, fit1_b1, fit1_w2, fit1_b2):
    params = {
        'embedding': [
            {'w0': emb0_w0, 'b0': emb0_b0, 'w1': emb0_w1, 'b1': emb0_b1},
            {'w0': emb1_w0, 'b0': emb1_b0, 'w1': emb1_w1, 'b1': emb1_b1},
        ],
        'fitting': [
            {'w0': fit0_w0, 'b0': fit0_b0, 'w1': fit0_w1, 'b1': fit0_b1,
             'w2': fit0_w2, 'b2': fit0_b2},
            {'w0': fit1_w0, 'b0': fit1_b0, 'w1': fit1_w1, 'b1': fit1_b1,
             'w2': fit1_w2, 'b2': fit1_b2},
        ],
    }
    return _forward(Ri, (2048, 2048), params)


# speedup vs baseline: 1.7261x; 1.0227x over previous
import functools

import jax
import jax.numpy as jnp
from jax.experimental import pallas as pl
from jax.experimental.pallas import tpu as pltpu

NT = 2
M = 32
EH = 16
EE = 32
FH = 32
TA = 128
TPS = 2
NNEI = NT * M
NC = NNEI * 4
SEG = M * TA
SCALE = 1.0 / (M * NT)
DT = jnp.float32


def _tile_kernel(type_ids_ref,
                 rif_ref,
                 ew0_ref, eb0_ref, ew1_ref, eb1_ref,
                 fw0_ref, fb0_ref, fw1_ref, fb1_ref, fw2_ref, fb2_ref,
                 ei_ref,
                 rt_buf, s_buf, g_buf, scat_buf, dr_buf):
    del type_ids_ref

    for tile in range(TPS):
        rt_buf[tile] = rif_ref[tile].T.astype(jnp.float32)

        for t1 in range(NT):
            for n in range(M):
                row = 4 * (M * t1 + n)
                s_buf[tile, t1:t1 + 1, n * TA:(n + 1) * TA] = \
                    rt_buf[tile, row:row + 1, :]

        for t1 in range(NT):
            s = s_buf[tile, t1:t1 + 1, :]
            h1 = jnp.tanh(ew0_ref[0, t1] * s + eb0_ref[0, t1])
            g = jnp.tanh(
                jnp.dot(ew1_ref[0, t1], h1.astype(jnp.bfloat16),
                        preferred_element_type=jnp.float32) + eb1_ref[0, t1])
            g_buf[tile, t1] = g + jnp.concatenate([h1, h1], axis=0)

        scat = tuple(jnp.zeros((EE, TA), jnp.float32) for _ in range(4))
        for t1 in range(NT):
            for n in range(M):
                gb = g_buf[tile, t1, :, n * TA:(n + 1) * TA]
                row = 4 * (M * t1 + n)
                r = rt_buf[tile, row:row + 4, :]
                scat = tuple(scat[c] + gb * r[c:c + 1, :] for c in range(4))

        sb = []
        for c in range(4):
            scat_buf[tile, c * EE:(c + 1) * EE, :] = scat[c]
            sb.append(scat[c][:EH, :])

        for e in range(EE):
            acc = scat_buf[tile, e:e + 1, :] * sb[0]
            for c in range(1, 4):
                acc = acc + scat_buf[tile, c * EE + e:c * EE + e + 1, :] * sb[c]
            dr_buf[tile, e * EH:(e + 1) * EH, :] = acc.astype(jnp.bfloat16)

        dr = dr_buf[tile]
        f1 = jnp.tanh(
            jnp.dot(fw0_ref[0], dr, preferred_element_type=jnp.float32)
            + fb0_ref[0])
        f2 = jnp.tanh(
            jnp.dot(fw1_ref[0], f1.astype(jnp.bfloat16),
                    preferred_element_type=jnp.float32) + fb1_ref[0]) + f1
        ei = jnp.sum(f2 * fw2_ref[0], axis=0, keepdims=True) + fb2_ref[0]
        ei_ref[tile] = ei.reshape(1, TA).astype(ei_ref.dtype)


def _pack_params(params):
    emb, fit = params['embedding'], params['fitting']
    ew0 = jnp.stack([jnp.transpose(emb[t]['w0'], (0, 2, 1)) for t in range(NT)])
    eb0 = jnp.stack([jnp.transpose(emb[t]['b0'], (0, 2, 1)) for t in range(NT)])
    ew1 = jnp.stack([jnp.transpose(emb[t]['w1'], (0, 2, 1))
                     for t in range(NT)]).astype(jnp.bfloat16)
    eb1 = jnp.stack([jnp.transpose(emb[t]['b1'], (0, 2, 1)) for t in range(NT)])
    fw0 = jnp.stack([(SCALE * SCALE) * fit[t]['w0'].T
                     for t in range(NT)]).astype(jnp.bfloat16)
    fb0 = jnp.stack([fit[t]['b0'].T for t in range(NT)])
    fw1 = jnp.stack([fit[t]['w1'].T for t in range(NT)]).astype(jnp.bfloat16)
    fb1 = jnp.stack([fit[t]['b1'].T for t in range(NT)])
    fw2 = jnp.stack([fit[t]['w2'] for t in range(NT)])
    fb2 = jnp.stack([fit[t]['b2'] for t in range(NT)])
    return ew0, eb0, ew1, eb1, fw0, fb0, fw1, fb1, fw2, fb2


def _run(type_ids, rif, weights, n_steps):
    def wspec(shape):
        nd = len(shape)
        return pl.BlockSpec((1,) + tuple(shape[1:]),
                            lambda i, tt, _nd=nd: (tt[i],) + (0,) * (_nd - 1))

    in_specs = [pl.BlockSpec((TPS, TA, NC), lambda i, tt: (i, 0, 0))]
    in_specs += [wspec(w.shape) for w in weights]

    return pl.pallas_call(
        _tile_kernel,
        out_shape=jax.ShapeDtypeStruct((n_steps * TPS, 1, TA), DT),
        grid_spec=pltpu.PrefetchScalarGridSpec(
            num_scalar_prefetch=1,
            grid=(n_steps,),
            in_specs=in_specs,
            out_specs=pl.BlockSpec((TPS, 1, TA), lambda i, tt: (i, 0, 0)),
            scratch_shapes=[
                pltpu.VMEM((TPS, NC, TA), jnp.float32),
                pltpu.VMEM((TPS, NT, SEG), jnp.float32),
                pltpu.VMEM((TPS, NT, EE, SEG), jnp.float32),
                pltpu.VMEM((TPS, 4 * EE, TA), jnp.float32),
                pltpu.VMEM((TPS, EH * EE, TA), jnp.bfloat16),
            ],
        ),
        compiler_params=pltpu.CompilerParams(
            dimension_semantics=("parallel",),
            vmem_limit_bytes=32 * 1024 * 1024,
        ),
    )(type_ids, rif, *weights)


@functools.partial(jax.jit, static_argnums=(1,))
def _forward(Ri, natoms, params):
    B = Ri.shape[0]
    natoms_sum = sum(natoms)
    rows = B * natoms_sum
    n_steps = rows // (TA * TPS)
    rif = Ri.reshape(rows // TA, TA, NC).astype(jnp.bfloat16)

    weights = _pack_params(params)

    type_ids = []
    for _ in range(B):
        for t in range(NT):
            type_ids += [t] * (natoms[t] // (TA * TPS))
    type_ids = jnp.asarray(type_ids, jnp.int32)

    ei_raw = _run(type_ids, rif, weights, n_steps)

    Ei = ei_raw.reshape(B, natoms_sum)
    Etot = jnp.sum(Ei, axis=1, keepdims=True)
    F = jnp.zeros((B, natoms_sum, 3), DT)
    return Etot, Ei, F


def kernel(Ri,
           emb0_w0, emb0_b0, emb0_w1, emb0_b1,
           emb1_w0, emb1_b0, emb1_w1, emb1_b1,
           fit0_w0, fit0_b0, fit0_w1, fit0_b1, fit0_w2, fit0_b2,
           fit1_w0, fit1_b0, fit1_w1, fit1_b1, fit1_w2, fit1_b2):
    params = {
        'embedding': [
            {'w0': emb0_w0, 'b0': emb0_b0, 'w1': emb0_w1, 'b1': emb0_b1},
            {'w0': emb1_w0, 'b0': emb1_b0, 'w1': emb1_w1, 'b1': emb1_b1},
        ],
        'fitting': [
            {'w0': fit0_w0, 'b0': fit0_b0, 'w1': fit0_w1, 'b1': fit0_b1,
             'w2': fit0_w2, 'b2': fit0_b2},
            {'w0': fit1_w0, 'b0': fit1_b0, 'w1': fit1_w1, 'b1': fit1_b1,
             'w2': fit1_w2, 'b2': fit1_b2},
        ],
    }
    return _forward(Ri, (2048, 2048), params)


# speedup vs baseline: 1.8526x; 1.0733x over previous
import functools

import jax
import jax.numpy as jnp
from jax.experimental import pallas as pl
from jax.experimental.pallas import tpu as pltpu

NT = 2
M = 32
EH = 16
EE = 32
FH = 32
TA = 128
TPS = 4
NNEI = NT * M
NC = NNEI * 4
SEG = M * TA
SCALE = 1.0 / (M * NT)
DT = jnp.float32


def _tile_kernel(type_ids_ref,
                 rif_ref,
                 ew0_ref, eb0_ref, ew1_ref, eb1_ref,
                 fw0_ref, fb0_ref, fw1_ref, fb1_ref, fw2_ref, fb2_ref,
                 ei_ref,
                 rt_buf, s_buf, g_buf, scat_buf, dr_buf):
    del type_ids_ref

    for tile in range(TPS):
        rt_buf[tile] = rif_ref[tile].T.astype(jnp.float32)

        for t1 in range(NT):
            for n in range(M):
                row = 4 * (M * t1 + n)
                s_buf[tile, t1:t1 + 1, n * TA:(n + 1) * TA] = \
                    rt_buf[tile, row:row + 1, :]

        for t1 in range(NT):
            s = s_buf[tile, t1:t1 + 1, :]
            h1 = jnp.tanh(ew0_ref[0, t1] * s + eb0_ref[0, t1])
            g = jnp.tanh(
                jnp.dot(ew1_ref[0, t1], h1.astype(jnp.bfloat16),
                        preferred_element_type=jnp.float32) + eb1_ref[0, t1])
            g_buf[tile, t1] = g + jnp.concatenate([h1, h1], axis=0)

        scat = tuple(jnp.zeros((EE, TA), jnp.float32) for _ in range(4))
        for t1 in range(NT):
            for n in range(M):
                gb = g_buf[tile, t1, :, n * TA:(n + 1) * TA]
                row = 4 * (M * t1 + n)
                r = rt_buf[tile, row:row + 4, :]
                scat = tuple(scat[c] + gb * r[c:c + 1, :] for c in range(4))

        sb = []
        for c in range(4):
            scat_buf[tile, c * EE:(c + 1) * EE, :] = scat[c]
            sb.append(scat[c][:EH, :])

        for e in range(EE):
            acc = scat_buf[tile, e:e + 1, :] * sb[0]
            for c in range(1, 4):
                acc = acc + scat_buf[tile, c * EE + e:c * EE + e + 1, :] * sb[c]
            dr_buf[tile, e * EH:(e + 1) * EH, :] = acc.astype(jnp.bfloat16)

        dr = dr_buf[tile]
        f1 = jnp.tanh(
            jnp.dot(fw0_ref[0], dr, preferred_element_type=jnp.float32)
            + fb0_ref[0])
        f2 = jnp.tanh(
            jnp.dot(fw1_ref[0], f1.astype(jnp.bfloat16),
                    preferred_element_type=jnp.float32) + fb1_ref[0]) + f1
        ei = jnp.sum(f2 * fw2_ref[0], axis=0, keepdims=True) + fb2_ref[0]
        ei_ref[tile] = ei.reshape(1, TA).astype(ei_ref.dtype)


def _pack_params(params):
    emb, fit = params['embedding'], params['fitting']
    ew0 = jnp.stack([jnp.transpose(emb[t]['w0'], (0, 2, 1)) for t in range(NT)])
    eb0 = jnp.stack([jnp.transpose(emb[t]['b0'], (0, 2, 1)) for t in range(NT)])
    ew1 = jnp.stack([jnp.transpose(emb[t]['w1'], (0, 2, 1))
                     for t in range(NT)]).astype(jnp.bfloat16)
    eb1 = jnp.stack([jnp.transpose(emb[t]['b1'], (0, 2, 1)) for t in range(NT)])
    fw0 = jnp.stack([(SCALE * SCALE) * fit[t]['w0'].T
                     for t in range(NT)]).astype(jnp.bfloat16)
    fb0 = jnp.stack([fit[t]['b0'].T for t in range(NT)])
    fw1 = jnp.stack([fit[t]['w1'].T for t in range(NT)]).astype(jnp.bfloat16)
    fb1 = jnp.stack([fit[t]['b1'].T for t in range(NT)])
    fw2 = jnp.stack([fit[t]['w2'] for t in range(NT)])
    fb2 = jnp.stack([fit[t]['b2'] for t in range(NT)])
    return ew0, eb0, ew1, eb1, fw0, fb0, fw1, fb1, fw2, fb2


def _run(type_ids, rif, weights, n_steps):
    def wspec(shape):
        nd = len(shape)
        return pl.BlockSpec((1,) + tuple(shape[1:]),
                            lambda i, tt, _nd=nd: (tt[i],) + (0,) * (_nd - 1))

    in_specs = [pl.BlockSpec((TPS, TA, NC), lambda i, tt: (i, 0, 0))]
    in_specs += [wspec(w.shape) for w in weights]

    return pl.pallas_call(
        _tile_kernel,
        out_shape=jax.ShapeDtypeStruct((n_steps * TPS, 1, TA), DT),
        grid_spec=pltpu.PrefetchScalarGridSpec(
            num_scalar_prefetch=1,
            grid=(n_steps,),
            in_specs=in_specs,
            out_specs=pl.BlockSpec((TPS, 1, TA), lambda i, tt: (i, 0, 0)),
            scratch_shapes=[
                pltpu.VMEM((TPS, NC, TA), jnp.float32),
                pltpu.VMEM((TPS, NT, SEG), jnp.float32),
                pltpu.VMEM((TPS, NT, EE, SEG), jnp.float32),
                pltpu.VMEM((TPS, 4 * EE, TA), jnp.float32),
                pltpu.VMEM((TPS, EH * EE, TA), jnp.bfloat16),
            ],
        ),
        compiler_params=pltpu.CompilerParams(
            dimension_semantics=("parallel",),
            vmem_limit_bytes=32 * 1024 * 1024,
        ),
    )(type_ids, rif, *weights)


@functools.partial(jax.jit, static_argnums=(1,))
def _forward(Ri, natoms, params):
    B = Ri.shape[0]
    natoms_sum = sum(natoms)
    rows = B * natoms_sum
    n_steps = rows // (TA * TPS)
    rif = Ri.reshape(rows // TA, TA, NC).astype(jnp.bfloat16)

    weights = _pack_params(params)

    type_ids = []
    for _ in range(B):
        for t in range(NT):
            type_ids += [t] * (natoms[t] // (TA * TPS))
    type_ids = jnp.asarray(type_ids, jnp.int32)

    ei_raw = _run(type_ids, rif, weights, n_steps)

    Ei = ei_raw.reshape(B, natoms_sum)
    Etot = jnp.sum(Ei, axis=1, keepdims=True)
    F = jnp.zeros((B, natoms_sum, 3), DT)
    return Etot, Ei, F


def kernel(Ri,
           emb0_w0, emb0_b0, emb0_w1, emb0_b1,
           emb1_w0, emb1_b0, emb1_w1, emb1_b1,
           fit0_w0, fit0_b0, fit0_w1, fit0_b1, fit0_w2, fit0_b2,
           fit1_w0, fit1_b0, fit1_w1, fit1_b1, fit1_w2, fit1_b2):
    params = {
        'embedding': [
            {'w0': emb0_w0, 'b0': emb0_b0, 'w1': emb0_w1, 'b1': emb0_b1},
            {'w0': emb1_w0, 'b0': emb1_b0, 'w1': emb1_w1, 'b1': emb1_b1},
        ],
        'fitting': [
            {'w0': fit0_w0, 'b0': fit0_b0, 'w1': fit0_w1, 'b1': fit0_b1,
             'w2': fit0_w2, 'b2': fit0_b2},
            {'w0': fit1_w0, 'b0': fit1_b0, 'w1': fit1_w1, 'b1': fit1_b1,
             'w2': fit1_w2, 'b2': fit1_b2},
        ],
    }
    return _forward(Ri, (2048, 2048), params)


# speedup vs baseline: 2.1867x; 1.1803x over previous
import functools

import jax
import jax.numpy as jnp
from jax.experimental import pallas as pl
from jax.experimental.pallas import tpu as pltpu

NT = 2
M = 32
EH = 16
EE = 32
FH = 32
TA = 128
TPS = 4
NNEI = NT * M
NC = NNEI * 4
SEG = M * TA
SCALE = 1.0 / (M * NT)
DT = jnp.float32


def _tile_kernel(type_ids_ref,
                 rif_ref,
                 ew0_ref, eb0_ref, ew1_ref, eb1_ref,
                 fw0_ref, fb0_ref, fw1_ref, fb1_ref, fw2_ref, fb2_ref,
                 ei_ref,
                 rt_buf, s_buf, g_buf, scat_buf, dr_buf):
    del type_ids_ref

    for tile in range(TPS):
        blk = rif_ref[0, tile * TA:(tile + 1) * TA, :]
        rt_buf[tile] = blk.T.astype(jnp.float32)

        for t1 in range(NT):
            for n in range(M):
                row = 4 * (M * t1 + n)
                s_buf[tile, t1:t1 + 1, n * TA:(n + 1) * TA] = \
                    rt_buf[tile, row:row + 1, :]

        for t1 in range(NT):
            s = s_buf[tile, t1:t1 + 1, :]
            h1 = jnp.tanh(ew0_ref[0, t1] * s + eb0_ref[0, t1])
            g = jnp.tanh(
                jnp.dot(ew1_ref[0, t1], h1.astype(jnp.bfloat16),
                        preferred_element_type=jnp.float32) + eb1_ref[0, t1])
            g_buf[tile, t1] = g + jnp.concatenate([h1, h1], axis=0)

        scat = tuple(jnp.zeros((EE, TA), jnp.float32) for _ in range(4))
        for t1 in range(NT):
            for n in range(M):
                gb = g_buf[tile, t1, :, n * TA:(n + 1) * TA]
                row = 4 * (M * t1 + n)
                r = rt_buf[tile, row:row + 4, :]
                scat = tuple(scat[c] + gb * r[c:c + 1, :] for c in range(4))

        sb = []
        for c in range(4):
            scat_buf[tile, c * EE:(c + 1) * EE, :] = scat[c]
            sb.append(scat[c][:EH, :])

        for e in range(EE):
            acc = scat_buf[tile, e:e + 1, :] * sb[0]
            for c in range(1, 4):
                acc = acc + scat_buf[tile, c * EE + e:c * EE + e + 1, :] * sb[c]
            dr_buf[tile, e * EH:(e + 1) * EH, :] = acc.astype(jnp.bfloat16)

        dr = dr_buf[tile]
        f1 = jnp.tanh(
            jnp.dot(fw0_ref[0], dr, preferred_element_type=jnp.float32)
            + fb0_ref[0])
        f2 = jnp.tanh(
            jnp.dot(fw1_ref[0], f1.astype(jnp.bfloat16),
                    preferred_element_type=jnp.float32) + fb1_ref[0]) + f1
        ei = jnp.sum(f2 * fw2_ref[0], axis=0, keepdims=True) + fb2_ref[0]
        ei_ref[tile] = ei.reshape(1, TA).astype(ei_ref.dtype)


def _pack_params(params):
    emb, fit = params['embedding'], params['fitting']
    ew0 = jnp.stack([jnp.transpose(emb[t]['w0'], (0, 2, 1)) for t in range(NT)])
    eb0 = jnp.stack([jnp.transpose(emb[t]['b0'], (0, 2, 1)) for t in range(NT)])
    ew1 = jnp.stack([jnp.transpose(emb[t]['w1'], (0, 2, 1))
                     for t in range(NT)]).astype(jnp.bfloat16)
    eb1 = jnp.stack([jnp.transpose(emb[t]['b1'], (0, 2, 1)) for t in range(NT)])
    fw0 = jnp.stack([(SCALE * SCALE) * fit[t]['w0'].T
                     for t in range(NT)]).astype(jnp.bfloat16)
    fb0 = jnp.stack([fit[t]['b0'].T for t in range(NT)])
    fw1 = jnp.stack([fit[t]['w1'].T for t in range(NT)]).astype(jnp.bfloat16)
    fb1 = jnp.stack([fit[t]['b1'].T for t in range(NT)])
    fw2 = jnp.stack([fit[t]['w2'] for t in range(NT)])
    fb2 = jnp.stack([fit[t]['b2'] for t in range(NT)])
    return ew0, eb0, ew1, eb1, fw0, fb0, fw1, fb1, fw2, fb2


def _run(type_ids, rif, weights, n_steps):
    def wspec(shape):
        nd = len(shape)
        return pl.BlockSpec((1,) + tuple(shape[1:]),
                            lambda i, tt, _nd=nd: (tt[i],) + (0,) * (_nd - 1))

    cpb = 4096 // (TPS * TA)
    in_specs = [pl.BlockSpec((1, TPS * TA, NC),
                             lambda i, tt: (i // cpb, i % cpb, 0))]
    in_specs += [wspec(w.shape) for w in weights]

    return pl.pallas_call(
        _tile_kernel,
        out_shape=jax.ShapeDtypeStruct((n_steps * TPS, 1, TA), DT),
        grid_spec=pltpu.PrefetchScalarGridSpec(
            num_scalar_prefetch=1,
            grid=(n_steps,),
            in_specs=in_specs,
            out_specs=pl.BlockSpec((TPS, 1, TA), lambda i, tt: (i, 0, 0)),
            scratch_shapes=[
                pltpu.VMEM((TPS, NC, TA), jnp.float32),
                pltpu.VMEM((TPS, NT, SEG), jnp.float32),
                pltpu.VMEM((TPS, NT, EE, SEG), jnp.float32),
                pltpu.VMEM((TPS, 4 * EE, TA), jnp.float32),
                pltpu.VMEM((TPS, EH * EE, TA), jnp.bfloat16),
            ],
        ),
        compiler_params=pltpu.CompilerParams(
            dimension_semantics=("parallel",),
            vmem_limit_bytes=32 * 1024 * 1024,
        ),
    )(type_ids, rif, *weights)


@functools.partial(jax.jit, static_argnums=(1,))
def _forward(Ri, natoms, params):
    B = Ri.shape[0]
    natoms_sum = sum(natoms)
    rows = B * natoms_sum
    n_steps = rows // (TA * TPS)
    rif = Ri.reshape(B, natoms_sum, NC).astype(jnp.bfloat16)

    weights = _pack_params(params)

    type_ids = []
    for _ in range(B):
        for t in range(NT):
            type_ids += [t] * (natoms[t] // (TA * TPS))
    type_ids = jnp.asarray(type_ids, jnp.int32)

    ei_raw = _run(type_ids, rif, weights, n_steps)

    Ei = ei_raw.reshape(B, natoms_sum)
    Etot = jnp.sum(Ei, axis=1, keepdims=True)
    F = jnp.zeros((B, natoms_sum, 3), DT)
    return Etot, Ei, F


def kernel(Ri,
           emb0_w0, emb0_b0, emb0_w1, emb0_b1,
           emb1_w0, emb1_b0, emb1_w1, emb1_b1,
           fit0_w0, fit0_b0, fit0_w1, fit0_b1, fit0_w2, fit0_b2,
           fit1_w0, fit1_b0, fit1_w1, fit1_b1, fit1_w2, fit1_b2):
    params = {
        'embedding': [
            {'w0': emb0_w0, 'b0': emb0_b0, 'w1': emb0_w1, 'b1': emb0_b1},
            {'w0': emb1_w0, 'b0': emb1_b0, 'w1': emb1_w1, 'b1': emb1_b1},
        ],
        'fitting': [
            {'w0': fit0_w0, 'b0': fit0_b0, 'w1': fit0_w1, 'b1': fit0_b1,
             'w2': fit0_w2, 'b2': fit0_b2},
            {'w0': fit1_w0, 'b0': fit1_b0, 'w1': fit1_w1, 'b1': fit1_b1,
             'w2': fit1_w2, 'b2': fit1_b2},
        ],
    }
    return _forward(Ri, (2048, 2048), params)


# speedup vs baseline: 2.2765x; 1.0411x over previous
import functools

import jax
import jax.numpy as jnp
from jax.experimental import pallas as pl
from jax.experimental.pallas import tpu as pltpu

NT = 2
M = 32
EH = 16
EE = 32
FH = 32
TA = 128
TPS = 8
NNEI = NT * M
NC = NNEI * 4
SEG = M * TA
SCALE = 1.0 / (M * NT)
DT = jnp.float32


def _tile_kernel(type_ids_ref,
                 rif_ref,
                 ew0_ref, eb0_ref, ew1_ref, eb1_ref,
                 fw0_ref, fb0_ref, fw1_ref, fb1_ref, fw2_ref, fb2_ref,
                 ei_ref,
                 rt_buf, s_buf, g_buf, scat_buf, dr_buf):
    del type_ids_ref

    for tile in range(TPS):
        blk = rif_ref[0, tile * TA:(tile + 1) * TA, :]
        rt_buf[tile] = blk.T.astype(jnp.float32)

        for t1 in range(NT):
            for n in range(M):
                row = 4 * (M * t1 + n)
                s_buf[tile, t1:t1 + 1, n * TA:(n + 1) * TA] = \
                    rt_buf[tile, row:row + 1, :]

        for t1 in range(NT):
            s = s_buf[tile, t1:t1 + 1, :]
            h1 = jnp.tanh(ew0_ref[0, t1] * s + eb0_ref[0, t1])
            g = jnp.tanh(
                jnp.dot(ew1_ref[0, t1], h1.astype(jnp.bfloat16),
                        preferred_element_type=jnp.float32) + eb1_ref[0, t1])
            g_buf[tile, t1] = g + jnp.concatenate([h1, h1], axis=0)

        scat = tuple(jnp.zeros((EE, TA), jnp.float32) for _ in range(4))
        for t1 in range(NT):
            for n in range(M):
                gb = g_buf[tile, t1, :, n * TA:(n + 1) * TA]
                row = 4 * (M * t1 + n)
                r = rt_buf[tile, row:row + 4, :]
                scat = tuple(scat[c] + gb * r[c:c + 1, :] for c in range(4))

        sb = []
        for c in range(4):
            scat_buf[tile, c * EE:(c + 1) * EE, :] = scat[c]
            sb.append(scat[c][:EH, :])

        for e in range(EE):
            acc = scat_buf[tile, e:e + 1, :] * sb[0]
            for c in range(1, 4):
                acc = acc + scat_buf[tile, c * EE + e:c * EE + e + 1, :] * sb[c]
            dr_buf[tile, e * EH:(e + 1) * EH, :] = acc.astype(jnp.bfloat16)

        dr = dr_buf[tile]
        f1 = jnp.tanh(
            jnp.dot(fw0_ref[0], dr, preferred_element_type=jnp.float32)
            + fb0_ref[0])
        f2 = jnp.tanh(
            jnp.dot(fw1_ref[0], f1.astype(jnp.bfloat16),
                    preferred_element_type=jnp.float32) + fb1_ref[0]) + f1
        ei = jnp.sum(f2 * fw2_ref[0], axis=0, keepdims=True) + fb2_ref[0]
        ei_ref[tile] = ei.reshape(1, TA).astype(ei_ref.dtype)


def _pack_params(params):
    emb, fit = params['embedding'], params['fitting']
    ew0 = jnp.stack([jnp.transpose(emb[t]['w0'], (0, 2, 1)) for t in range(NT)])
    eb0 = jnp.stack([jnp.transpose(emb[t]['b0'], (0, 2, 1)) for t in range(NT)])
    ew1 = jnp.stack([jnp.transpose(emb[t]['w1'], (0, 2, 1))
                     for t in range(NT)]).astype(jnp.bfloat16)
    eb1 = jnp.stack([jnp.transpose(emb[t]['b1'], (0, 2, 1)) for t in range(NT)])
    fw0 = jnp.stack([(SCALE * SCALE) * fit[t]['w0'].T
                     for t in range(NT)]).astype(jnp.bfloat16)
    fb0 = jnp.stack([fit[t]['b0'].T for t in range(NT)])
    fw1 = jnp.stack([fit[t]['w1'].T for t in range(NT)]).astype(jnp.bfloat16)
    fb1 = jnp.stack([fit[t]['b1'].T for t in range(NT)])
    fw2 = jnp.stack([fit[t]['w2'] for t in range(NT)])
    fb2 = jnp.stack([fit[t]['b2'] for t in range(NT)])
    return ew0, eb0, ew1, eb1, fw0, fb0, fw1, fb1, fw2, fb2


def _run(type_ids, rif, weights, n_steps):
    def wspec(shape):
        nd = len(shape)
        return pl.BlockSpec((1,) + tuple(shape[1:]),
                            lambda i, tt, _nd=nd: (tt[i],) + (0,) * (_nd - 1))

    cpb = 4096 // (TPS * TA)
    in_specs = [pl.BlockSpec((1, TPS * TA, NC),
                             lambda i, tt: (i // cpb, i % cpb, 0))]
    in_specs += [wspec(w.shape) for w in weights]

    return pl.pallas_call(
        _tile_kernel,
        out_shape=jax.ShapeDtypeStruct((n_steps * TPS, 1, TA), DT),
        grid_spec=pltpu.PrefetchScalarGridSpec(
            num_scalar_prefetch=1,
            grid=(n_steps,),
            in_specs=in_specs,
            out_specs=pl.BlockSpec((TPS, 1, TA), lambda i, tt: (i, 0, 0)),
            scratch_shapes=[
                pltpu.VMEM((TPS, NC, TA), jnp.float32),
                pltpu.VMEM((TPS, NT, SEG), jnp.float32),
                pltpu.VMEM((TPS, NT, EE, SEG), jnp.float32),
                pltpu.VMEM((TPS, 4 * EE, TA), jnp.float32),
                pltpu.VMEM((TPS, EH * EE, TA), jnp.bfloat16),
            ],
        ),
        compiler_params=pltpu.CompilerParams(
            dimension_semantics=("parallel",),
            vmem_limit_bytes=32 * 1024 * 1024,
        ),
    )(type_ids, rif, *weights)


@functools.partial(jax.jit, static_argnums=(1,))
def _forward(Ri, natoms, params):
    B = Ri.shape[0]
    natoms_sum = sum(natoms)
    rows = B * natoms_sum
    n_steps = rows // (TA * TPS)
    rif = Ri.reshape(B, natoms_sum, NC).astype(jnp.bfloat16)

    weights = _pack_params(params)

    type_ids = []
    for _ in range(B):
        for t in range(NT):
            type_ids += [t] * (natoms[t] // (TA * TPS))
    type_ids = jnp.asarray(type_ids, jnp.int32)

    ei_raw = _run(type_ids, rif, weights, n_steps)

    Ei = ei_raw.reshape(B, natoms_sum)
    Etot = jnp.sum(Ei, axis=1, keepdims=True)
    F = jnp.zeros((B, natoms_sum, 3), DT)
    return Etot, Ei, F


def kernel(Ri,
           emb0_w0, emb0_b0, emb0_w1, emb0_b1,
           emb1_w0, emb1_b0, emb1_w1, emb1_b1,
           fit0_w0, fit0_b0, fit0_w1, fit0_b1, fit0_w2, fit0_b2,
           fit1_w0, fit1_b0, fit1_w1, fit1_b1, fit1_w2, fit1_b2):
    params = {
        'embedding': [
            {'w0': emb0_w0, 'b0': emb0_b0, 'w1': emb0_w1, 'b1': emb0_b1},
            {'w0': emb1_w0, 'b0': emb1_b0, 'w1': emb1_w1, 'b1': emb1_b1},
        ],
        'fitting': [
            {'w0': fit0_w0, 'b0': fit0_b0, 'w1': fit0_w1, 'b1': fit0_b1,
             'w2': fit0_w2, 'b2': fit0_b2},
            {'w0': fit1_w0, 'b0': fit1_b0, 'w1': fit1_w1, 'b1': fit1_b1,
             'w2': fit1_w2, 'b2': fit1_b2},
        ],
    }
    return _forward(Ri, (2048, 2048), params)
